# baseline (speedup 1.0000x reference)
import jax
import jax.numpy as jnp
from jax.experimental import pallas as pl
from jax.experimental.pallas import tpu as pltpu
from jax.experimental.pallas import tpu_sc as plsc
import functools

B = 4096
FEAT = 128
HD = 10000
WT = 2048
T = 5
HDP = T * WT
NC8 = 8
NWORK = 32
BPW = B // NWORK

_BN_INV = 1.0 / (1.0 + 1e-5) ** 0.5
_SN_SCALE = 1.0 / (float(HD) ** 0.5 + 1e-8)


def _encoder_kernel(data_ref,
                    w1_ref, b1_ref, g1_ref, be1_ref,
                    w2_ref, b2_ref, g2_ref, be2_ref,
                    w3_ref, b3_ref, g3_ref, be3_ref,
                    feat_ref):
    h = jnp.dot(data_ref[...], w1_ref[...], preferred_element_type=jnp.float32)
    h = jnp.maximum(g1_ref[...] * ((h + b1_ref[...]) * _BN_INV) + be1_ref[...], 0.0)
    h = jnp.dot(h, w2_ref[...], preferred_element_type=jnp.float32)
    h = jnp.maximum(g2_ref[...] * ((h + b2_ref[...]) * _BN_INV) + be2_ref[...], 0.0)
    h = jnp.dot(h, w3_ref[...], preferred_element_type=jnp.float32)
    feat_ref[...] = jnp.maximum(
        g3_ref[...] * ((h + b3_ref[...]) * _BN_INV) + be3_ref[...],
        0.0).astype(jnp.bfloat16)


def _onehot_sc_body(lab_hbm, out_hbm, lab_v, oh_v):
    wid = jax.lax.axis_index("s") * 2 + jax.lax.axis_index("c")
    base = wid * BPW
    pltpu.sync_copy(lab_hbm.at[pl.ds(base, BPW)], lab_v)
    for k in range(BPW // 16):
        lab16 = lab_v[pl.ds(k * 16, 16)]
        for cc in range(NC8):
            oh_v[cc, pl.ds(k * 16, 16)] = jnp.where(
                lab16 == cc, 1.0, 0.0).astype(jnp.float32)
    pltpu.sync_copy(oh_v, out_hbm.at[:, pl.ds(base, BPW)])


_onehot_sc = functools.partial(
    pl.kernel,
    mesh=plsc.VectorSubcoreMesh(core_axis_name="c", subcore_axis_name="s"),
    out_type=jax.ShapeDtypeStruct((NC8, B), jnp.float32),
    scratch_types=[
        pltpu.VMEM((BPW,), jnp.int32),
        pltpu.VMEM((NC8, BPW), jnp.float32),
    ],
)(_onehot_sc_body)


def _main_kernel(feat_ref, oht_ref, proj_ref, out_ref,
                 sgn_ref, acc_ref, sims_ref, gram_ref):
    j = pl.program_id(0)

    @pl.when(j < T)
    def _produce():
        ones = jnp.full((B, 128), 0x3F80, jnp.int16)
        sbit = jnp.full((B, 128), -0x8000, jnp.int16)
        for c in range(WT // 128):
            hvc = jnp.dot(feat_ref[...], proj_ref[:, c * 128:(c + 1) * 128],
                          preferred_element_type=jnp.float32).astype(jnp.bfloat16)
            bits = jax.lax.bitcast_convert_type(hvc, jnp.int16)
            sgn_ref[j % 2, :, c * 128:(c + 1) * 128] = (
                jax.lax.bitcast_convert_type((bits & sbit) | ones,
                                             jnp.bfloat16))
        acc_ref[j % 2] = jnp.dot(oht_ref[...].astype(jnp.bfloat16), sgn_ref[j % 2],
                                 preferred_element_type=jnp.float32)

    @pl.when(j > 0)
    def _consume():
        t = j - 1
        signed = sgn_ref[(j + 1) % 2]
        col = jax.lax.broadcasted_iota(jnp.int32, (1, WT), 1) + t * WT
        protos = jnp.where(col < HD, jnp.sign(acc_ref[(j + 1) % 2]),
                           0.0).astype(jnp.bfloat16)
        protos_t = protos.T
        part = jnp.dot(signed, protos_t, preferred_element_type=jnp.float32)
        gpart = jnp.dot(protos, protos_t, preferred_element_type=jnp.float32)

        @pl.when(t == 0)
        def _init():
            sims_ref[...] = part
            gram_ref[...] = gpart

        @pl.when(t > 0)
        def _acc():
            sims_ref[...] += part
            gram_ref[...] += gpart

        @pl.when(t == T - 1)
        def _finalize():
            pnormsq = jnp.sum(
                gram_ref[...] * jnp.eye(NC8, dtype=jnp.float32),
                axis=0, keepdims=True)
            scale = _SN_SCALE / (jnp.sqrt(pnormsq) + 1e-8)
            out_ref[...] = sims_ref[...] * scale


def kernel(data, labels, W1, b1, g1, be1, W2, b2, g2, be2, W3, b3, g3, be3,
           proj):
    row = lambda v: v.reshape(1, -1)
    oht = _onehot_sc(labels.astype(jnp.int32))
    feats = pl.pallas_call(
        _encoder_kernel,
        out_shape=jax.ShapeDtypeStruct((B, FEAT), jnp.bfloat16),
    )(data, W1, row(b1), row(g1), row(be1),
      W2, row(b2), row(g2), row(be2), W3, row(b3), row(g3), row(be3))

    projp = proj.astype(jnp.bfloat16)
    out = pl.pallas_call(
        _main_kernel,
        grid=(T + 1,),
        in_specs=[
            pl.BlockSpec((B, FEAT), lambda j: (0, 0)),
            pl.BlockSpec((NC8, B), lambda j: (0, 0)),
            pl.BlockSpec((FEAT, WT), lambda j: (0, jnp.minimum(j, T - 1))),
        ],
        out_specs=pl.BlockSpec((B, NC8), lambda j: (0, 0)),
        out_shape=jax.ShapeDtypeStruct((B, NC8), jnp.float32),
        scratch_shapes=[
            pltpu.VMEM((2, B, WT), jnp.bfloat16),
            pltpu.VMEM((2, NC8, WT), jnp.float32),
            pltpu.VMEM((B, NC8), jnp.float32),
            pltpu.VMEM((NC8, NC8), jnp.float32),
        ],
        compiler_params=pltpu.CompilerParams(
            dimension_semantics=("arbitrary",)),
    )(feats, oht, projp)
    return out[:, :6]


# speedup vs baseline: 1.1876x; 1.1876x over previous
import jax
import jax.numpy as jnp
from jax.experimental import pallas as pl
from jax.experimental.pallas import tpu as pltpu

B = 4096
FEAT = 128
HD = 10000
WT = 2048
T = 5
HDP = T * WT
NC8 = 8

_BN_INV = 1.0 / (1.0 + 1e-5) ** 0.5
_SN_SCALE = 1.0 / (float(HD) ** 0.5 + 1e-8)


def _encoder_kernel(data_ref, lab_ref,
                    w1_ref, b1_ref, g1_ref, be1_ref,
                    w2_ref, b2_ref, g2_ref, be2_ref,
                    w3_ref, b3_ref, g3_ref, be3_ref,
                    feat_ref, oht_ref):
    h = jnp.dot(data_ref[...], w1_ref[...], preferred_element_type=jnp.float32)
    h = jnp.maximum(g1_ref[...] * ((h + b1_ref[...]) * _BN_INV) + be1_ref[...], 0.0)
    h = jnp.dot(h, w2_ref[...], preferred_element_type=jnp.float32)
    h = jnp.maximum(g2_ref[...] * ((h + b2_ref[...]) * _BN_INV) + be2_ref[...], 0.0)
    h = jnp.dot(h, w3_ref[...], preferred_element_type=jnp.float32)
    feat_ref[...] = jnp.maximum(
        g3_ref[...] * ((h + b3_ref[...]) * _BN_INV) + be3_ref[...],
        0.0).astype(jnp.bfloat16)
    cls = jax.lax.broadcasted_iota(jnp.int32, (NC8, B), 0)
    oht_ref[...] = (cls == lab_ref[...]).astype(jnp.bfloat16)


def _main_kernel(feat_ref, oht_ref, proj_ref, out_ref,
                 sgn_ref, acc_ref, sims_ref, gram_ref):
    j = pl.program_id(0)

    @pl.when(j < T)
    def _produce():
        ones = jnp.full((B, 128), 0x3F80, jnp.int16)
        sbit = jnp.full((B, 128), -0x8000, jnp.int16)
        for c in range(WT // 128):
            hvc = jnp.dot(feat_ref[...],
                          proj_ref[:, c * 128:(c + 1) * 128].astype(jnp.bfloat16),
                          preferred_element_type=jnp.float32).astype(jnp.bfloat16)
            bits = jax.lax.bitcast_convert_type(hvc, jnp.int16)
            sgn_ref[j % 2, :, c * 128:(c + 1) * 128] = (
                jax.lax.bitcast_convert_type((bits & sbit) | ones,
                                             jnp.bfloat16))
        acc_ref[j % 2] = jnp.dot(oht_ref[...], sgn_ref[j % 2],
                                 preferred_element_type=jnp.float32)

    @pl.when(j > 0)
    def _consume():
        t = j - 1
        signed = sgn_ref[(j + 1) % 2]
        col = jax.lax.broadcasted_iota(jnp.int32, (1, WT), 1) + t * WT
        protos = jnp.where(col < HD, jnp.sign(acc_ref[(j + 1) % 2]),
                           0.0).astype(jnp.bfloat16)
        protos_t = protos.T
        part = jnp.dot(signed, protos_t, preferred_element_type=jnp.float32)
        gpart = jnp.dot(protos, protos_t, preferred_element_type=jnp.float32)

        @pl.when(t == 0)
        def _init():
            sims_ref[...] = part
            gram_ref[...] = gpart

        @pl.when(t > 0)
        def _acc():
            sims_ref[...] += part
            gram_ref[...] += gpart

        @pl.when(t == T - 1)
        def _finalize():
            pnormsq = jnp.sum(
                gram_ref[...] * jnp.eye(NC8, dtype=jnp.float32),
                axis=0, keepdims=True)
            scale = _SN_SCALE / (jnp.sqrt(pnormsq) + 1e-8)
            out_ref[...] = (sims_ref[...] * scale)[:, :6]


def kernel(data, labels, W1, b1, g1, be1, W2, b2, g2, be2, W3, b3, g3, be3,
           proj):
    lab = labels.astype(jnp.int32).reshape(1, B)
    row = lambda v: v.reshape(1, -1)
    feats, oht = pl.pallas_call(
        _encoder_kernel,
        out_shape=[
            jax.ShapeDtypeStruct((B, FEAT), jnp.bfloat16),
            jax.ShapeDtypeStruct((NC8, B), jnp.bfloat16),
        ],
    )(data, lab, W1, row(b1), row(g1), row(be1),
      W2, row(b2), row(g2), row(be2), W3, row(b3), row(g3), row(be3))

    projp = proj
    out = pl.pallas_call(
        _main_kernel,
        grid=(T + 1,),
        in_specs=[
            pl.BlockSpec((B, FEAT), lambda j: (0, 0)),
            pl.BlockSpec((NC8, B), lambda j: (0, 0)),
            pl.BlockSpec((FEAT, WT), lambda j: (0, jnp.minimum(j, T - 1))),
        ],
        out_specs=pl.BlockSpec((B, 6), lambda j: (0, 0)),
        out_shape=jax.ShapeDtypeStruct((B, 6), jnp.float32),
        scratch_shapes=[
            pltpu.VMEM((2, B, WT), jnp.bfloat16),
            pltpu.VMEM((2, NC8, WT), jnp.float32),
            pltpu.VMEM((B, NC8), jnp.float32),
            pltpu.VMEM((NC8, NC8), jnp.float32),
        ],
        compiler_params=pltpu.CompilerParams(
            dimension_semantics=("arbitrary",)),
    )(feats, oht, projp)
    return out
